# baseline (speedup 1.0000x reference)
import functools
import math

import jax
import jax.numpy as jnp
from jax import lax
from jax.experimental import pallas as pl
from jax.experimental.pallas import tpu as pltpu
from jax.experimental.pallas import tpu_sc as plsc

D_MODEL = 64
SCALE = math.sqrt(D_MODEL)
LANES = 16
IDX_ROW = 128
ROWS_PER_CHUNK = 4
CHUNK = IDX_ROW * ROWS_PER_CHUNK


def _make_kernel(n_total):
    info = plsc.get_sparse_core_info()
    nw = info.num_cores * info.num_subcores
    per_w = n_total // nw
    n_chunks = per_w // CHUNK
    assert per_w % CHUNK == 0

    mesh = plsc.VectorSubcoreMesh(core_axis_name="c", subcore_axis_name="s")

    @functools.partial(
        pl.kernel,
        mesh=mesh,
        out_type=jax.ShapeDtypeStruct((n_total, D_MODEL), jnp.float32),
        scratch_types=[
            pltpu.VMEM((ROWS_PER_CHUNK, IDX_ROW), jnp.int32),
            pltpu.VMEM((CHUNK, D_MODEL), jnp.float32),
            pltpu.SemaphoreType.DMA,
        ],
        compiler_params=pltpu.CompilerParams(use_tc_tiling_on_sc=False),
    )
    def emb(x_hbm, lut_hbm, out_hbm, idx_v, rows_v, sem):
        wid = lax.axis_index("s") * info.num_cores + lax.axis_index("c")
        idx_row_base = wid * (per_w // IDX_ROW)
        out_base = wid * per_w

        def chunk_body(g, _):
            pltpu.sync_copy(
                x_hbm.at[pl.ds(idx_row_base + g * ROWS_PER_CHUNK, ROWS_PER_CHUNK)],
                idx_v,
            )
            copies = [
                pltpu.async_copy(
                    lut_hbm.at[idx_v.at[j]],
                    rows_v.at[pl.ds(j * IDX_ROW, IDX_ROW)],
                    sem,
                )
                for j in range(ROWS_PER_CHUNK)
            ]
            for c in copies:
                c.wait()

            def scale_body(i, _):
                for j in range(D_MODEL // LANES):
                    v = rows_v[i, pl.ds(j * LANES, LANES)]
                    rows_v[i, pl.ds(j * LANES, LANES)] = v * SCALE
                return 0

            lax.fori_loop(0, CHUNK, scale_body, 0)
            pltpu.sync_copy(
                rows_v, out_hbm.at[pl.ds(out_base + g * CHUNK, CHUNK)]
            )
            return 0

        lax.fori_loop(0, n_chunks, chunk_body, 0)

    return emb


def kernel(x, lut):
    b, s = x.shape
    n = b * s
    xi = x.reshape(n // IDX_ROW, IDX_ROW).astype(jnp.int32)
    out = _make_kernel(n)(xi, lut)
    return out.reshape(b, s, D_MODEL)


# speedup vs baseline: 1.1147x; 1.1147x over previous
import functools
import math

import jax
import jax.numpy as jnp
from jax import lax
from jax.experimental import pallas as pl
from jax.experimental.pallas import tpu as pltpu
from jax.experimental.pallas import tpu_sc as plsc

D_MODEL = 64
SCALE = math.sqrt(D_MODEL)
LANES = 16
IDX_ROW = 128
ROWS_PER_CHUNK = 4
CHUNK = IDX_ROW * ROWS_PER_CHUNK


def _make_kernel(n_total):
    info = plsc.get_sparse_core_info()
    nw = info.num_cores * info.num_subcores
    per_w = n_total // nw
    n_chunks = per_w // CHUNK
    assert per_w % CHUNK == 0 and n_chunks % 2 == 0

    mesh = plsc.VectorSubcoreMesh(core_axis_name="c", subcore_axis_name="s")

    @functools.partial(
        pl.kernel,
        mesh=mesh,
        out_type=jax.ShapeDtypeStruct((n_total, D_MODEL), jnp.float32),
        scratch_types=[
            pltpu.VMEM((2, ROWS_PER_CHUNK, IDX_ROW), jnp.int32),
            pltpu.VMEM((CHUNK, D_MODEL), jnp.float32),
            pltpu.VMEM((CHUNK, D_MODEL), jnp.float32),
            pltpu.SemaphoreType.DMA,
            pltpu.SemaphoreType.DMA,
        ],
        compiler_params=pltpu.CompilerParams(use_tc_tiling_on_sc=False),
    )
    def emb(x_hbm, lut_hbm, out_hbm, idx_v, rows0_v, rows1_v, sem0, sem1):
        wid = lax.axis_index("s") * info.num_cores + lax.axis_index("c")
        idx_row_base = wid * (per_w // IDX_ROW)
        out_base = wid * per_w
        rows = (rows0_v, rows1_v)
        sems = (sem0, sem1)

        def start_gathers(g, buf):
            pltpu.sync_copy(
                x_hbm.at[pl.ds(idx_row_base + g * ROWS_PER_CHUNK, ROWS_PER_CHUNK)],
                idx_v.at[buf],
            )
            for j in range(ROWS_PER_CHUNK):
                pltpu.async_copy(
                    lut_hbm.at[idx_v.at[buf, j]],
                    rows[buf].at[pl.ds(j * IDX_ROW, IDX_ROW)],
                    sems[buf],
                )

        def drain_gathers(buf):
            for j in range(ROWS_PER_CHUNK):
                pltpu.make_async_copy(
                    lut_hbm.at[idx_v.at[buf, j]],
                    rows[buf].at[pl.ds(j * IDX_ROW, IDX_ROW)],
                    sems[buf],
                ).wait()

        start_gathers(0, 0)

        def pair_body(g2, _):
            for b in range(2):
                g = 2 * g2 + b
                p, q = b, 1 - b
                drain_gathers(p)

                @pl.when(g + 1 < n_chunks)
                def _():
                    start_gathers(g + 1, q)

                @plsc.parallel_loop(0, CHUNK, step=1, unroll=8)
                def _(i):
                    for j in range(D_MODEL // LANES):
                        v = rows[p][i, pl.ds(j * LANES, LANES)]
                        rows[p][i, pl.ds(j * LANES, LANES)] = v * SCALE

                pltpu.sync_copy(
                    rows[p], out_hbm.at[pl.ds(out_base + g * CHUNK, CHUNK)]
                )
            return 0

        lax.fori_loop(0, n_chunks // 2, pair_body, 0)

    return emb


def kernel(x, lut):
    b, s = x.shape
    n = b * s
    xi = x.reshape(n // IDX_ROW, IDX_ROW).astype(jnp.int32)
    out = _make_kernel(n)(xi, lut)
    return out.reshape(b, s, D_MODEL)


# speedup vs baseline: 1.2644x; 1.1343x over previous
import functools
import math

import jax
import jax.numpy as jnp
from jax import lax
from jax.experimental import pallas as pl
from jax.experimental.pallas import tpu as pltpu
from jax.experimental.pallas import tpu_sc as plsc

D_MODEL = 64
SCALE = math.sqrt(D_MODEL)
LANES = 16
IDX_ROW = 128
ROWS_PER_CHUNK = 2
CHUNK = IDX_ROW * ROWS_PER_CHUNK


def _make_kernel(n_total):
    info = plsc.get_sparse_core_info()
    nw = info.num_cores * info.num_subcores
    per_w = n_total // nw
    n_chunks = per_w // CHUNK
    assert per_w % CHUNK == 0 and n_chunks % 2 == 0

    mesh = plsc.VectorSubcoreMesh(core_axis_name="c", subcore_axis_name="s")

    @functools.partial(
        pl.kernel,
        mesh=mesh,
        out_type=jax.ShapeDtypeStruct((n_total, D_MODEL), jnp.float32),
        scratch_types=[
            pltpu.VMEM((2, ROWS_PER_CHUNK, IDX_ROW), jnp.int32),
            pltpu.VMEM((2, ROWS_PER_CHUNK, IDX_ROW), jnp.int32),
            pltpu.VMEM((CHUNK, 2 * D_MODEL), jnp.float32),
            pltpu.VMEM((CHUNK, 2 * D_MODEL), jnp.float32),
            pltpu.VMEM((CHUNK, D_MODEL), jnp.float32),
            pltpu.SemaphoreType.DMA,
            pltpu.SemaphoreType.DMA,
        ],
        compiler_params=pltpu.CompilerParams(needs_layout_passes=False),
    )
    def emb(x_hbm, lut2_hbm, out_hbm, idx_v, gidx_v, g0_v, g1_v, stg_v,
            sem0, sem1):
        wid = lax.axis_index("s") * info.num_cores + lax.axis_index("c")
        idx_row_base = wid * (per_w // IDX_ROW)
        out_base = wid * per_w
        gath = (g0_v, g1_v)
        sems = (sem0, sem1)
        iota = lax.iota(jnp.int32, LANES)

        def start_gathers(g, buf):
            pltpu.sync_copy(
                x_hbm.at[pl.ds(idx_row_base + g * ROWS_PER_CHUNK, ROWS_PER_CHUNK)],
                idx_v.at[buf],
            )
            for j in range(ROWS_PER_CHUNK):
                for k in range(IDX_ROW // LANES):
                    ids = idx_v[buf, j, pl.ds(k * LANES, LANES)]
                    gidx_v[buf, j, pl.ds(k * LANES, LANES)] = ids >> 1
            for j in range(ROWS_PER_CHUNK):
                pltpu.async_copy(
                    lut2_hbm.at[gidx_v.at[buf, j]],
                    gath[buf].at[pl.ds(j * IDX_ROW, IDX_ROW)],
                    sems[buf],
                )

        def drain_gathers(buf):
            for j in range(ROWS_PER_CHUNK):
                pltpu.make_async_copy(
                    lut2_hbm.at[gidx_v.at[buf, j]],
                    gath[buf].at[pl.ds(j * IDX_ROW, IDX_ROW)],
                    sems[buf],
                ).wait()

        start_gathers(0, 0)

        def pair_body(g2, _):
            for b in range(2):
                g = 2 * g2 + b
                p, q = b, 1 - b
                drain_gathers(p)

                @pl.when(g + 1 < n_chunks)
                def _():
                    start_gathers(g + 1, q)

                @plsc.parallel_loop(0, CHUNK, step=1, unroll=4)
                def _(i):
                    row = jnp.full((LANES,), i, jnp.int32)
                    ids = plsc.load_gather(
                        idx_v, [jnp.full((LANES,), p, jnp.int32),
                                row >> 7, jnp.full((LANES,), i & 127, jnp.int32)]
                    )
                    half = (ids & 1) << 6
                    for j in range(D_MODEL // LANES):
                        v = plsc.load_gather(gath[p], [row, half + (j * LANES) + iota])
                        stg_v[i, pl.ds(j * LANES, LANES)] = v * SCALE

                pltpu.sync_copy(
                    stg_v, out_hbm.at[pl.ds(out_base + g * CHUNK, CHUNK)]
                )
            return 0

        lax.fori_loop(0, n_chunks // 2, pair_body, 0)

    return emb


def kernel(x, lut):
    b, s = x.shape
    n = b * s
    xi = x.reshape(n // IDX_ROW, IDX_ROW).astype(jnp.int32)
    lut2 = lut.reshape(lut.shape[0] // 2, 2 * D_MODEL)
    out = _make_kernel(n)(xi, lut2)
    return out.reshape(b, s, D_MODEL)
